# baseline (speedup 1.0000x reference)
import jax
import jax.numpy as jnp
from jax import lax
from jax.experimental import pallas as pl
from jax.experimental.pallas import tpu as pltpu, tpu_sc as plsc

NC, NS = 2, 16
NW = NC * NS
B, L, D = 16384, 200, 32
BPW = B // NW
G = 16
CW = 128
CPG = G * L // CW
NG = BPW // G
UN = 8


def _body(data_hbm, table_hbm, out_hbm, idx_v, rows_v, out_v, sem):
    wid = lax.axis_index("s") * NC + lax.axis_index("c")
    base_row = wid * BPW
    base_chunk = base_row * L // CW

    @pl.loop(0, NG)
    def _(g):
        pltpu.sync_copy(data_hbm.at[pl.ds(base_chunk + g * CPG, CPG)], idx_v)
        descs = []
        for c in range(CPG):
            descs.append(
                pltpu.async_copy(
                    table_hbm.at[idx_v.at[c]],
                    rows_v.at[pl.ds(c * CW, CW)],
                    sem,
                )
            )
        for d in descs:
            d.wait()
        for r in range(G):
            e0 = r * L

            def rbody(i, accs):
                a0, a1 = accs
                e = e0 + i * UN
                for k in range(UN):
                    a0 = a0 + rows_v[e + k, 0:16]
                    a1 = a1 + rows_v[e + k, 16:32]
                return a0, a1

            z = jnp.zeros((16,), jnp.float32)
            a0, a1 = lax.fori_loop(0, L // UN, rbody, (z, z))
            out_v[r, 0:16] = a0
            out_v[r, 16:32] = a1
        pltpu.sync_copy(out_v, out_hbm.at[pl.ds(base_row + g * G, G)])


def kernel(data, lengths, table):
    del lengths
    data2 = data.reshape(B * L // CW, CW)
    mesh = plsc.VectorSubcoreMesh(core_axis_name="c", subcore_axis_name="s")
    f = pl.kernel(
        _body,
        out_type=jax.ShapeDtypeStruct((B, D), jnp.float32),
        mesh=mesh,
        scratch_types=[
            pltpu.VMEM((CPG, CW), jnp.int32),
            pltpu.VMEM((G * L, D), jnp.float32),
            pltpu.VMEM((G, D), jnp.float32),
            pltpu.SemaphoreType.DMA,
        ],
        compiler_params=pltpu.CompilerParams(use_tc_tiling_on_sc=False),
    )
    return f(data2, table)


# speedup vs baseline: 1.1232x; 1.1232x over previous
import jax
import jax.numpy as jnp
from jax import lax
from jax.experimental import pallas as pl
from jax.experimental.pallas import tpu as pltpu, tpu_sc as plsc

NC, NS = 2, 16
NW = NC * NS
B, L, D = 16384, 200, 32
BPW = B // NW
G = 8
NG = BPW // G
UN = 8
C0 = 128
C1 = L - C0


def _body(d0_hbm, d1_hbm, table_hbm, out_hbm, idx0_v, idx1_v, rows_v, out_v,
          sem0, sem1):
    wid = lax.axis_index("s") * NC + lax.axis_index("c")
    base_row = wid * BPW
    sems = (sem0, sem1)

    def fire(g, b):
        row0 = base_row + g * G
        pltpu.sync_copy(d0_hbm.at[pl.ds(row0, G)], idx0_v.at[b])
        pltpu.sync_copy(d1_hbm.at[pl.ds(row0, G)], idx1_v.at[b])
        for r in range(G):
            pltpu.async_copy(
                table_hbm.at[idx0_v.at[b, r]],
                rows_v.at[b, pl.ds(r * L, C0)],
                sems[b],
            )
            pltpu.async_copy(
                table_hbm.at[idx1_v.at[b, r, pl.ds(128 - C1, C1)]],
                rows_v.at[b, pl.ds(r * L + C0, C1)],
                sems[b],
            )

    def drain(b):
        pltpu.make_async_copy(
            table_hbm.at[pl.ds(0, G * L)], rows_v.at[b], sems[b]
        ).wait()

    def accum(g, b):
        for r in range(G):
            e0 = r * L

            def rbody(i, accs):
                a0, a1 = accs
                e = e0 + i * UN
                for k in range(UN):
                    a0 = a0 + rows_v[b, e + k, 0:16]
                    a1 = a1 + rows_v[b, e + k, 16:32]
                return a0, a1

            z = jnp.zeros((16,), jnp.float32)
            a0, a1 = lax.fori_loop(0, L // UN, rbody, (z, z))
            out_v[b, r, 0:16] = a0
            out_v[b, r, 16:32] = a1
        pltpu.sync_copy(out_v.at[b], out_hbm.at[pl.ds(base_row + g * G, G)])

    fire(0, 0)

    @pl.loop(0, NG - 2, step=2)
    def _(g):
        fire(g + 1, 1)
        drain(0)
        accum(g, 0)
        fire(g + 2, 0)
        drain(1)
        accum(g + 1, 1)

    fire(NG - 1, 1)
    drain(0)
    accum(NG - 2, 0)
    drain(1)
    accum(NG - 1, 1)


def kernel(data, lengths, table):
    del lengths
    d0 = lax.slice(data, (0, 0), (B, C0))
    d1 = lax.slice(data, (0, L - 128), (B, L))
    mesh = plsc.VectorSubcoreMesh(core_axis_name="c", subcore_axis_name="s")
    f = pl.kernel(
        _body,
        out_type=jax.ShapeDtypeStruct((B, D), jnp.float32),
        mesh=mesh,
        scratch_types=[
            pltpu.VMEM((2, G, 128), jnp.int32),
            pltpu.VMEM((2, G, 128), jnp.int32),
            pltpu.VMEM((2, G * L, D), jnp.float32),
            pltpu.VMEM((2, G, D), jnp.float32),
            pltpu.SemaphoreType.DMA,
            pltpu.SemaphoreType.DMA,
        ],
        compiler_params=pltpu.CompilerParams(use_tc_tiling_on_sc=False),
    )
    return f(d0, d1, table)


# speedup vs baseline: 1.1353x; 1.0108x over previous
import jax
import jax.numpy as jnp
from jax import lax
from jax.experimental import pallas as pl
from jax.experimental.pallas import tpu as pltpu, tpu_sc as plsc

NC, NS = 2, 16
VOCAB_N = 1000000
NW = NC * NS
B, L, D = 16384, 200, 32
BPW = B // NW
G = 8
NG = BPW // G
UN = 8
C0 = 128
C1 = L - C0


def _body(d0_hbm, d1_hbm, table_hbm, out_hbm, idx0_v, idx1_v, rows_v, out_v,
          sem0, sem1):
    wid = lax.axis_index("s") * NC + lax.axis_index("c")
    base_row = wid * BPW
    sems = (sem0, sem1)

    def fire(g, b):
        row0 = base_row + g * G
        pltpu.sync_copy(d0_hbm.at[pl.ds(row0, G)], idx0_v.at[b])
        pltpu.sync_copy(d1_hbm.at[pl.ds(row0, G)], idx1_v.at[b])
        for r in range(G):
            pltpu.async_copy(
                table_hbm.at[idx0_v.at[b, r]],
                rows_v.at[b, pl.ds(r * L, C0)],
                sems[b],
            )
            pltpu.async_copy(
                table_hbm.at[idx1_v.at[b, r, pl.ds(128 - C1, C1)]],
                rows_v.at[b, pl.ds(r * L + C0, C1)],
                sems[b],
            )

    def drain(b):
        pltpu.make_async_copy(
            table_hbm.at[pl.ds(0, G * L)], rows_v.at[b], sems[b]
        ).wait()

    def accum(g, b):
        for r in range(G):
            e0 = r * L

            def rbody(i, accs):
                a0, a1 = accs
                e = e0 + i * UN
                for k in range(UN):
                    a0 = a0 + rows_v[b, e + k, 0:16]
                    a1 = a1 + rows_v[b, e + k, 16:32]
                return a0, a1

            z = jnp.zeros((16,), jnp.float32)
            a0, a1 = lax.fori_loop(0, L // UN, rbody, (z, z))
            out_v[b, r, 0:16] = a0
            out_v[b, r, 16:32] = a1
        pltpu.sync_copy(out_v.at[b], out_hbm.at[pl.ds(base_row + g * G, G)])

    fire(0, 0)

    @pl.loop(0, NG - 2, step=2)
    def _(g):
        fire(g + 1, 1)
        drain(0)
        accum(g, 0)
        fire(g + 2, 0)
        drain(1)
        accum(g + 1, 1)

    fire(NG - 1, 1)
    drain(0)
    accum(NG - 2, 0)
    drain(1)
    accum(NG - 1, 1)


def kernel(data, lengths, table):
    del lengths
    d0 = lax.slice(data, (0, 0), (B, C0)) * 4
    d1 = lax.slice(data, (0, L - 128), (B, L)) * 4
    table_p = jnp.pad(table, ((0, 0), (0, 128 - D))).reshape(4 * VOCAB_N, D)
    mesh = plsc.VectorSubcoreMesh(core_axis_name="c", subcore_axis_name="s")
    f = pl.kernel(
        _body,
        out_type=jax.ShapeDtypeStruct((B, D), jnp.float32),
        mesh=mesh,
        scratch_types=[
            pltpu.VMEM((2, G, 128), jnp.int32),
            pltpu.VMEM((2, G, 128), jnp.int32),
            pltpu.VMEM((2, G * L, D), jnp.float32),
            pltpu.VMEM((2, G, D), jnp.float32),
            pltpu.SemaphoreType.DMA,
            pltpu.SemaphoreType.DMA,
        ],
        compiler_params=pltpu.CompilerParams(use_tc_tiling_on_sc=False),
    )
    return f(d0, d1, table_p)
